# baseline (speedup 1.0000x reference)
import functools

import jax
import jax.numpy as jnp
from jax import lax
from jax.experimental import pallas as pl
from jax.experimental.pallas import tpu as pltpu
from jax.experimental.pallas import tpu_sc as plsc

_K = 16
_MAXP = 512
_FMAX = 3.4e38
_B = 4
_M = 1024
_D = 9
_SEG = 128
_CHUNK = 256


def _sc_compact_body(t_hbm, m_hbm, out_hbm, cnt_hbm,
                     mloc, mseg, tloc, zbuf, idxbuf, cbuf):
    c = lax.axis_index("c")
    s = lax.axis_index("s")
    b = 2 * c + s // 8
    sib = s % 8
    rowbase = b * _M + sib * _SEG

    pltpu.sync_copy(m_hbm.at[pl.ds(b * _M, _M)], mloc)
    pltpu.sync_copy(m_hbm.at[pl.ds(rowbase, _SEG)], mseg)
    pltpu.sync_copy(t_hbm.at[pl.ds(rowbase * _D, _SEG * _D)], tloc)

    z16 = jnp.zeros((16,), jnp.float32)
    for i in range(_SEG):
        zbuf[i, 0:16] = z16
    pltpu.sync_copy(zbuf, out_hbm.at[pl.ds(c * 2 * _M + s * _SEG, _SEG)])
    plsc.subcore_barrier()

    lane = lax.iota(jnp.int32, 16)
    l15 = jnp.full((16,), 15, jnp.int32)

    def _ladder(x):
        for j in (1, 2, 4, 8):
            sh = x.at[jnp.maximum(lane - j, 0)].get(mode="promise_in_bounds")
            x = x + jnp.where(lane >= j, sh, 0)
        return x

    pacc = jnp.zeros((16,), jnp.int32)
    for k2 in range(_M // 16):
        pacc = pacc + (mloc[k2 * 16:(k2 + 1) * 16] *
                       jnp.int32(k2 < sib * (_SEG // 16)))
    prefix = _ladder(pacc).at[l15].get(mode="promise_in_bounds")

    trash = jnp.full((16,), _B * _M, jnp.int32)
    for i in range(_SEG // 16):
        idxbuf[pl.ds(i * 16, 16)] = trash

    segoff = jnp.zeros((16,), jnp.int32)
    for g in range(_SEG // 16):
        mv = mseg[g * 16:(g + 1) * 16]
        mb = mv > 0
        csum = _ladder(mv)
        lpos = segoff + csum - mv
        plsc.store_scatter(idxbuf, [lpos], b * _M + prefix + lpos, mask=mb)
        for dd in range(_D):
            gidx = (g * 16 + lane) * _D + dd
            v = plsc.load_gather(tloc, [gidx])
            plsc.store_scatter(
                zbuf, [lpos, jnp.full((16,), dd, jnp.int32)], v, mask=mb)
        segoff = segoff + csum.at[l15].get(mode="promise_in_bounds")

    pltpu.sync_copy(zbuf, out_hbm.at[idxbuf])

    for i in range(8):
        cv = jnp.where((lane == 0) & (i == 0), segoff.astype(jnp.float32), 0.0)
        cbuf[pl.ds(i * 16, 16)] = cv
    pltpu.sync_copy(cbuf, cnt_hbm.at[c * 16 + s])


@functools.lru_cache(maxsize=1)
def _get_sc_compact():
    return pl.kernel(
        _sc_compact_body,
        mesh=plsc.VectorSubcoreMesh(core_axis_name="c", subcore_axis_name="s"),
        compiler_params=pltpu.CompilerParams(needs_layout_passes=False),
        out_type=[
            jax.ShapeDtypeStruct((_B * _M + 1, 128), jnp.float32),
            jax.ShapeDtypeStruct((32, 128), jnp.float32),
        ],
        scratch_types=[
            pltpu.VMEM((_M,), jnp.int32),
            pltpu.VMEM((_SEG,), jnp.int32),
            pltpu.VMEM((_SEG * _D,), jnp.float32),
            pltpu.VMEM((_SEG, 128), jnp.float32),
            pltpu.VMEM((_SEG,), jnp.int32),
            pltpu.VMEM((128,), jnp.float32),
        ],
    )


def _to_key(s, axis):
    b = lax.bitcast_convert_type(s, jnp.int32)
    ridx = lax.broadcasted_iota(jnp.int32, s.shape, axis)
    return lax.bitcast_convert_type((b & -512) | ridx, jnp.float32)


def _bottom_k_sum(s, k, axis):
    key = _to_key(s, axis)
    acc = jnp.zeros([1 if i == axis else d for i, d in enumerate(s.shape)],
                    jnp.float32)
    for _ in range(k):
        mkey = jnp.min(key, axis=axis, keepdims=True)
        acc = acc + mkey
        key = jnp.where(key == mkey, _FMAX, key)
    return acc


def _gj_inverse(a, n):
    ri = lax.broadcasted_iota(jnp.int32, (n, 1), 0)
    for kk in range(n):
        piv = a[kk:kk + 1, kk:kk + 1]
        rowk = a[kk:kk + 1, :] / piv
        colk = a[:, kk:kk + 1]
        a = jnp.where(ri == kk, rowk, a - colk * rowk)
    return a


def _loss_body(tc_ref, segc_ref, p_ref, pt_ref, oi_ref, oo_ref, acc_ref):
    tcm = tc_ref[...]
    segc = segc_ref[...]
    p = p_ref[...]
    pt = pt_ref[...]

    d = _D
    n = pt.shape[1]
    b = _B
    m = _M
    t = tcm[:, :d]

    counts = [jnp.sum(segc[8 * bi:8 * bi + 8, 0:1]) for bi in range(b)]
    count = counts[0] + counts[1] + counts[2] + counts[3]

    ri_m = lax.broadcasted_iota(jnp.int32, (m, 1), 0)
    valid = jnp.concatenate(
        [(ri_m < counts[bi].astype(jnp.int32)).astype(jnp.float32)
         for bi in range(b)], axis=0)
    mean = jnp.sum(t, axis=0, keepdims=True) / count
    am = (t - mean) * valid
    cov = lax.dot_general(
        am, am, (((0,), (0,)), ((), ())),
        preferred_element_type=jnp.float32) / (count - 1.0)
    ri = lax.broadcasted_iota(jnp.int32, (d, 1), 0)
    ci = lax.broadcasted_iota(jnp.int32, (1, d), 1)
    eye = (ci == ri).astype(jnp.float32)
    cinv = _gj_inverse(jnp.concatenate([cov, eye], axis=1), d)[:, d:]
    meant = jnp.sum(eye * mean, axis=1, keepdims=True)

    gp = jnp.dot(p, cinv, preferred_element_type=jnp.float32)
    beta = jnp.sum(gp * p, axis=1, keepdims=True)
    gpaug = jnp.concatenate([-2.0 * gp, beta], axis=1)

    outer_acc = jnp.float32(0.0)
    kf = jnp.float32(_K)
    acc_ref[0, 0] = jnp.float32(0.0)
    for bi in range(b):
        gpb = gpaug[bi * n:(bi + 1) * n, :]
        ptb = pt[bi * d:(bi + 1) * d, :]
        ctb = ptb + meant
        cct = jnp.dot(cinv, ctb, preferred_element_type=jnp.float32)
        gamma = jnp.sum(ctb * cct, axis=0, keepdims=True)
        ctaug = jnp.concatenate(
            [ctb, jnp.ones((1, n), jnp.float32)], axis=0)
        sot = jnp.dot(gpb, ctaug, preferred_element_type=jnp.float32)
        acc_o = _bottom_k_sum(sot, _K, 0) + kf * gamma
        outer_acc = outer_acc + jnp.sum(acc_o)

        gpt = jnp.dot(cinv, ptb, preferred_element_type=jnp.float32)
        betat = jnp.sum(ptb * gpt, axis=0, keepdims=True)
        gpaugt = jnp.concatenate([-2.0 * gpt, betat], axis=0)
        cnt_i = counts[bi].astype(jnp.int32)
        for ch in range(m // _CHUNK):

            @pl.when(ch * _CHUNK < cnt_i)
            def _():
                base = bi * m + ch * _CHUNK
                ach = t[base:base + _CHUNK, :] - mean
                gach = jnp.dot(ach, cinv,
                               preferred_element_type=jnp.float32)
                alpha = jnp.sum(gach * ach, axis=1, keepdims=True)
                ataug = jnp.concatenate(
                    [ach, jnp.ones((_CHUNK, 1), jnp.float32)], axis=1)
                sch = jnp.dot(ataug, gpaugt,
                              preferred_element_type=jnp.float32)
                accch = _bottom_k_sum(sch, _K, 1) + kf * alpha
                ri_ch = lax.broadcasted_iota(jnp.int32, (_CHUNK, 1), 0)
                vch = (ch * _CHUNK + ri_ch < cnt_i).astype(jnp.float32)
                acc_ref[0, 0] = acc_ref[0, 0] + jnp.sum(accch * vch)

    oi_ref[0, 0] = acc_ref[0, 0] / count
    oo_ref[0, 0] = outer_acc / jnp.float32(b * n * _K)


def kernel(outputs, targets, mask):
    bsz = targets.shape[0]
    msz = targets.shape[1]
    d = targets.shape[2]
    y_pred = outputs[:, :_MAXP]
    nsz = y_pred.shape[1]

    t_flat1d = targets.reshape(bsz * msz * d)
    mask32 = mask.reshape(bsz * msz).astype(jnp.int32)
    comp, segc = _get_sc_compact()(t_flat1d, mask32)
    tcomp = comp[:bsz * msz]

    p_flat = y_pred.reshape(bsz * nsz, d)
    p_t = y_pred.transpose(0, 2, 1).reshape(bsz * d, nsz)

    intra, outer = pl.pallas_call(
        _loss_body,
        out_shape=[
            jax.ShapeDtypeStruct((1, 1), jnp.float32),
            jax.ShapeDtypeStruct((1, 1), jnp.float32),
        ],
        out_specs=[
            pl.BlockSpec(memory_space=pltpu.SMEM),
            pl.BlockSpec(memory_space=pltpu.SMEM),
        ],
        scratch_shapes=[pltpu.SMEM((1, 1), jnp.float32)],
    )(tcomp, segc, p_flat, p_t)

    intra_loss = intra[0, 0]
    outer_loss = outer[0, 0]
    return (intra_loss, intra_loss, outer_loss)


# speedup vs baseline: 1.0086x; 1.0086x over previous
import functools

import jax
import jax.numpy as jnp
from jax import lax
from jax.experimental import pallas as pl
from jax.experimental.pallas import tpu as pltpu
from jax.experimental.pallas import tpu_sc as plsc

_K = 16
_MAXP = 512
_FMAX = 3.4e38
_B = 4
_M = 1024
_D = 9
_SEG = 128
_CHUNK = 256


def _sc_compact_body(t_hbm, m_hbm, out_hbm, cnt_hbm,
                     mloc, mseg, tloc, zbuf, idxbuf, cbuf):
    c = lax.axis_index("c")
    s = lax.axis_index("s")
    b = 2 * c + s // 8
    sib = s % 8
    rowbase = b * _M + sib * _SEG

    pltpu.sync_copy(m_hbm.at[pl.ds(b * _M, _M)], mloc)
    pltpu.sync_copy(m_hbm.at[pl.ds(rowbase, _SEG)], mseg)
    pltpu.sync_copy(t_hbm.at[pl.ds(rowbase * _D, _SEG * _D)], tloc)


    lane = lax.iota(jnp.int32, 16)
    l15 = jnp.full((16,), 15, jnp.int32)

    def _ladder(x):
        for j in (1, 2, 4, 8):
            sh = x.at[jnp.maximum(lane - j, 0)].get(mode="promise_in_bounds")
            x = x + jnp.where(lane >= j, sh, 0)
        return x

    pacc = jnp.zeros((16,), jnp.int32)
    for k2 in range(_M // 16):
        pacc = pacc + (mloc[k2 * 16:(k2 + 1) * 16] *
                       jnp.int32(k2 < sib * (_SEG // 16)))
    prefix = _ladder(pacc).at[l15].get(mode="promise_in_bounds")

    trash = jnp.full((16,), _B * _M, jnp.int32)
    for i in range(_SEG // 16):
        idxbuf[pl.ds(i * 16, 16)] = trash

    segoff = jnp.zeros((16,), jnp.int32)
    for g in range(_SEG // 16):
        mv = mseg[g * 16:(g + 1) * 16]
        mb = mv > 0
        csum = _ladder(mv)
        lpos = segoff + csum - mv
        plsc.store_scatter(idxbuf, [lpos], b * _M + prefix + lpos, mask=mb)
        for dd in range(_D):
            gidx = (g * 16 + lane) * _D + dd
            v = plsc.load_gather(tloc, [gidx])
            plsc.store_scatter(
                zbuf, [lpos, jnp.full((16,), dd, jnp.int32)], v, mask=mb)
        segoff = segoff + csum.at[l15].get(mode="promise_in_bounds")

    pltpu.sync_copy(zbuf, out_hbm.at[idxbuf])

    for i in range(8):
        cv = jnp.where((lane == 0) & (i == 0), segoff.astype(jnp.float32), 0.0)
        cbuf[pl.ds(i * 16, 16)] = cv
    pltpu.sync_copy(cbuf, cnt_hbm.at[c * 16 + s])


@functools.lru_cache(maxsize=1)
def _get_sc_compact():
    return pl.kernel(
        _sc_compact_body,
        mesh=plsc.VectorSubcoreMesh(core_axis_name="c", subcore_axis_name="s"),
        compiler_params=pltpu.CompilerParams(needs_layout_passes=False),
        out_type=[
            jax.ShapeDtypeStruct((_B * _M + 1, 128), jnp.float32),
            jax.ShapeDtypeStruct((32, 128), jnp.float32),
        ],
        scratch_types=[
            pltpu.VMEM((_M,), jnp.int32),
            pltpu.VMEM((_SEG,), jnp.int32),
            pltpu.VMEM((_SEG * _D,), jnp.float32),
            pltpu.VMEM((_SEG, 128), jnp.float32),
            pltpu.VMEM((_SEG,), jnp.int32),
            pltpu.VMEM((128,), jnp.float32),
        ],
    )


def _to_key(s, axis):
    b = lax.bitcast_convert_type(s, jnp.int32)
    ridx = lax.broadcasted_iota(jnp.int32, s.shape, axis)
    return lax.bitcast_convert_type((b & -512) | ridx, jnp.float32)


def _bottom_k_sum(s, k, axis):
    key = _to_key(s, axis)
    acc = jnp.zeros([1 if i == axis else d for i, d in enumerate(s.shape)],
                    jnp.float32)
    for _ in range(k):
        mkey = jnp.min(key, axis=axis, keepdims=True)
        acc = acc + mkey
        key = jnp.where(key == mkey, _FMAX, key)
    return acc


def _gj_inverse(a, n):
    ri = lax.broadcasted_iota(jnp.int32, (n, 1), 0)
    for kk in range(n):
        piv = a[kk:kk + 1, kk:kk + 1]
        rowk = a[kk:kk + 1, :] / piv
        colk = a[:, kk:kk + 1]
        a = jnp.where(ri == kk, rowk, a - colk * rowk)
    return a


def _loss_body(tc_ref, segc_ref, p_ref, pt_ref, oi_ref, oo_ref, acc_ref):
    tcm = tc_ref[...]
    segc = segc_ref[...]
    p = p_ref[...]
    pt = pt_ref[...]

    d = _D
    n = pt.shape[1]
    b = _B
    m = _M
    t = tcm[:, :d]

    counts = [jnp.sum(segc[8 * bi:8 * bi + 8, 0:1]) for bi in range(b)]
    count = counts[0] + counts[1] + counts[2] + counts[3]

    ri_m = lax.broadcasted_iota(jnp.int32, (m, 1), 0)
    valid = jnp.concatenate(
        [(ri_m < counts[bi].astype(jnp.int32)).astype(jnp.float32)
         for bi in range(b)], axis=0)
    tmask = jnp.where(valid > 0, t, 0.0)
    mean = jnp.sum(tmask, axis=0, keepdims=True) / count
    am = jnp.where(valid > 0, t - mean, 0.0)
    cov = lax.dot_general(
        am, am, (((0,), (0,)), ((), ())),
        preferred_element_type=jnp.float32) / (count - 1.0)
    ri = lax.broadcasted_iota(jnp.int32, (d, 1), 0)
    ci = lax.broadcasted_iota(jnp.int32, (1, d), 1)
    eye = (ci == ri).astype(jnp.float32)
    cinv = _gj_inverse(jnp.concatenate([cov, eye], axis=1), d)[:, d:]
    meant = jnp.sum(eye * mean, axis=1, keepdims=True)

    gp = jnp.dot(p, cinv, preferred_element_type=jnp.float32)
    beta = jnp.sum(gp * p, axis=1, keepdims=True)
    gpaug = jnp.concatenate([-2.0 * gp, beta], axis=1)

    outer_acc = jnp.float32(0.0)
    kf = jnp.float32(_K)
    acc_ref[0, 0] = jnp.float32(0.0)
    for bi in range(b):
        gpb = gpaug[bi * n:(bi + 1) * n, :]
        ptb = pt[bi * d:(bi + 1) * d, :]
        ctb = ptb + meant
        cct = jnp.dot(cinv, ctb, preferred_element_type=jnp.float32)
        gamma = jnp.sum(ctb * cct, axis=0, keepdims=True)
        ctaug = jnp.concatenate(
            [ctb, jnp.ones((1, n), jnp.float32)], axis=0)
        sot = jnp.dot(gpb, ctaug, preferred_element_type=jnp.float32)
        acc_o = _bottom_k_sum(sot, _K, 0) + kf * gamma
        outer_acc = outer_acc + jnp.sum(acc_o)

        gpt = jnp.dot(cinv, ptb, preferred_element_type=jnp.float32)
        betat = jnp.sum(ptb * gpt, axis=0, keepdims=True)
        gpaugt = jnp.concatenate([-2.0 * gpt, betat], axis=0)
        cnt_i = counts[bi].astype(jnp.int32)
        for ch in range(m // _CHUNK):

            @pl.when(ch * _CHUNK < cnt_i)
            def _():
                base = bi * m + ch * _CHUNK
                ach = t[base:base + _CHUNK, :] - mean
                gach = jnp.dot(ach, cinv,
                               preferred_element_type=jnp.float32)
                alpha = jnp.sum(gach * ach, axis=1, keepdims=True)
                ataug = jnp.concatenate(
                    [ach, jnp.ones((_CHUNK, 1), jnp.float32)], axis=1)
                sch = jnp.dot(ataug, gpaugt,
                              preferred_element_type=jnp.float32)
                accch = _bottom_k_sum(sch, _K, 1) + kf * alpha
                ri_ch = lax.broadcasted_iota(jnp.int32, (_CHUNK, 1), 0)
                live = ch * _CHUNK + ri_ch < cnt_i
                acc_ref[0, 0] = acc_ref[0, 0] + jnp.sum(
                    jnp.where(live, accch, 0.0))

    oi_ref[0, 0] = acc_ref[0, 0] / count
    oo_ref[0, 0] = outer_acc / jnp.float32(b * n * _K)


def kernel(outputs, targets, mask):
    bsz = targets.shape[0]
    msz = targets.shape[1]
    d = targets.shape[2]
    y_pred = outputs[:, :_MAXP]
    nsz = y_pred.shape[1]

    t_flat1d = targets.reshape(bsz * msz * d)
    mask32 = mask.reshape(bsz * msz).astype(jnp.int32)
    comp, segc = _get_sc_compact()(t_flat1d, mask32)
    tcomp = comp[:bsz * msz]

    p_flat = y_pred.reshape(bsz * nsz, d)
    p_t = y_pred.transpose(0, 2, 1).reshape(bsz * d, nsz)

    intra, outer = pl.pallas_call(
        _loss_body,
        out_shape=[
            jax.ShapeDtypeStruct((1, 1), jnp.float32),
            jax.ShapeDtypeStruct((1, 1), jnp.float32),
        ],
        out_specs=[
            pl.BlockSpec(memory_space=pltpu.SMEM),
            pl.BlockSpec(memory_space=pltpu.SMEM),
        ],
        scratch_shapes=[pltpu.SMEM((1, 1), jnp.float32)],
    )(tcomp, segc, p_flat, p_t)

    intra_loss = intra[0, 0]
    outer_loss = outer[0, 0]
    return (intra_loss, intra_loss, outer_loss)


# speedup vs baseline: 3.2367x; 3.2090x over previous
import jax
import jax.numpy as jnp
from jax import lax
from jax.experimental import pallas as pl
from jax.experimental.pallas import tpu as pltpu

_K = 16
_MAXP = 512
_FMAX = 3.4e38


def _to_key(s):
    b = lax.bitcast_convert_type(s, jnp.int32)
    ridx = lax.broadcasted_iota(jnp.int32, s.shape, 0)
    return lax.bitcast_convert_type((b & -512) | ridx, jnp.float32)


def _bottom_k_colsum(s, k):
    cols = s.shape[1]
    key = _to_key(s)
    acc = jnp.zeros((1, cols), jnp.float32)
    for _ in range(k):
        mkey = jnp.min(key, axis=0, keepdims=True)
        acc = acc + mkey
        key = jnp.where(key == mkey, _FMAX, key)
    return acc


def _gj_inverse(a, n):
    ri = lax.broadcasted_iota(jnp.int32, (n, 1), 0)
    for kk in range(n):
        piv = a[kk:kk + 1, kk:kk + 1]
        rowk = a[kk:kk + 1, :] / piv
        colk = a[:, kk:kk + 1]
        a = jnp.where(ri == kk, rowk, a - colk * rowk)
    return a


def _loss_body(t_ref, mk_ref, tt_ref, p_ref, pt_ref, mrow_ref,
               oi_ref, oo_ref):
    t = t_ref[...]
    mk = mk_ref[...]
    tt = tt_ref[...]
    p = p_ref[...]
    pt = pt_ref[...]
    mrow = mrow_ref[...]

    bm = t.shape[0]
    bn = p.shape[0]
    d = t.shape[1]
    n = pt.shape[1]
    b = bn // n
    m = bm // b

    count = jnp.sum(mk)
    mean = jnp.sum(t * mk, axis=0, keepdims=True) / count
    am = (t - mean) * mk
    cov = lax.dot_general(
        am, am, (((0,), (0,)), ((), ())),
        preferred_element_type=jnp.float32) / (count - 1.0)
    ri = lax.broadcasted_iota(jnp.int32, (d, 1), 0)
    ci = lax.broadcasted_iota(jnp.int32, (1, d), 1)
    eye = (ci == ri).astype(jnp.float32)
    cinv = _gj_inverse(jnp.concatenate([cov, eye], axis=1), d)[:, d:]
    meant = jnp.sum(eye * mean, axis=1, keepdims=True)

    gp = jnp.dot(p, cinv, preferred_element_type=jnp.float32)
    beta = jnp.sum(gp * p, axis=1, keepdims=True)
    gpaug = jnp.concatenate([-2.0 * gp, beta], axis=1)

    intra_acc = jnp.float32(0.0)
    outer_acc = jnp.float32(0.0)
    kf = jnp.float32(_K)
    for bi in range(b):
        gpb = gpaug[bi * n:(bi + 1) * n, :]

        atb = tt[bi * d:(bi + 1) * d, :] - meant
        cat = jnp.dot(cinv, atb, preferred_element_type=jnp.float32)
        alpha = jnp.sum(atb * cat, axis=0, keepdims=True)
        ataug = jnp.concatenate(
            [atb, jnp.ones((1, atb.shape[1]), jnp.float32)], axis=0)
        sit = jnp.dot(gpb, ataug, preferred_element_type=jnp.float32)
        acc_i = _bottom_k_colsum(sit, _K) + kf * alpha
        intra_acc = intra_acc + jnp.sum(acc_i * mrow[bi:bi + 1, :])

        ctb = pt[bi * d:(bi + 1) * d, :] + meant
        cct = jnp.dot(cinv, ctb, preferred_element_type=jnp.float32)
        gamma = jnp.sum(ctb * cct, axis=0, keepdims=True)
        ctaug = jnp.concatenate(
            [ctb, jnp.ones((1, ctb.shape[1]), jnp.float32)], axis=0)
        sot = jnp.dot(gpb, ctaug, preferred_element_type=jnp.float32)
        acc_o = _bottom_k_colsum(sot, _K) + kf * gamma
        outer_acc = outer_acc + jnp.sum(acc_o)

    oi_ref[0, 0] = intra_acc / count
    oo_ref[0, 0] = outer_acc / jnp.float32(bn * _K)


def kernel(outputs, targets, mask):
    bsz = targets.shape[0]
    msz = targets.shape[1]
    d = targets.shape[2]
    y_pred = outputs[:, :_MAXP]
    nsz = y_pred.shape[1]

    t_flat = targets.reshape(bsz * msz, d)
    t_t = targets.transpose(0, 2, 1).reshape(bsz * d, msz)
    mk = mask.reshape(bsz * msz, 1).astype(jnp.float32)
    mrow = mask.astype(jnp.float32)
    p_flat = y_pred.reshape(bsz * nsz, d)
    p_t = y_pred.transpose(0, 2, 1).reshape(bsz * d, nsz)

    intra, outer = pl.pallas_call(
        _loss_body,
        out_shape=[
            jax.ShapeDtypeStruct((1, 1), jnp.float32),
            jax.ShapeDtypeStruct((1, 1), jnp.float32),
        ],
        out_specs=[
            pl.BlockSpec(memory_space=pltpu.SMEM),
            pl.BlockSpec(memory_space=pltpu.SMEM),
        ],
    )(t_flat, mk, t_t, p_flat, p_t, mrow)

    intra_loss = intra[0, 0]
    outer_loss = outer[0, 0]
    return (intra_loss, intra_loss, outer_loss)


# speedup vs baseline: 3.4253x; 1.0583x over previous
import jax
import jax.numpy as jnp
from jax import lax
from jax.experimental import pallas as pl
from jax.experimental.pallas import tpu as pltpu

_K = 16
_MAXP = 512
_FMAX = 3.4e38


def _to_key(s):
    b = lax.bitcast_convert_type(s, jnp.int32)
    ridx = lax.broadcasted_iota(jnp.int32, s.shape, 0)
    return lax.bitcast_convert_type((b & -512) | ridx, jnp.float32)


def _bottom_k_colsum(s, k):
    key = _to_key(s)
    m = jnp.min(key, axis=0, keepdims=True)
    acc = m
    for _ in range(k - 1):
        m = jnp.min(jnp.where(key > m, key, _FMAX), axis=0, keepdims=True)
        acc = acc + m
    return acc


def _gj_inverse(a, n):
    ri = lax.broadcasted_iota(jnp.int32, (n, 1), 0)
    for kk in range(n):
        piv = a[kk:kk + 1, kk:kk + 1]
        rowk = a[kk:kk + 1, :] / piv
        colk = a[:, kk:kk + 1]
        a = jnp.where(ri == kk, rowk, a - colk * rowk)
    return a


def _loss_body(t_ref, mk_ref, tt_ref, p_ref, pt_ref, mrow_ref,
               oi_ref, oo_ref):
    t = t_ref[...]
    mk = mk_ref[...]
    tt = tt_ref[...]
    p = p_ref[...]
    pt = pt_ref[...]
    mrow = mrow_ref[...]

    bm = t.shape[0]
    bn = p.shape[0]
    d = t.shape[1]
    n = pt.shape[1]
    b = bn // n
    m = bm // b

    count = jnp.sum(mk)
    mean = jnp.sum(t * mk, axis=0, keepdims=True) / count
    am = (t - mean) * mk
    cov = lax.dot_general(
        am, am, (((0,), (0,)), ((), ())),
        preferred_element_type=jnp.float32) / (count - 1.0)
    ri = lax.broadcasted_iota(jnp.int32, (d, 1), 0)
    ci = lax.broadcasted_iota(jnp.int32, (1, d), 1)
    eye = (ci == ri).astype(jnp.float32)
    cinv = _gj_inverse(jnp.concatenate([cov, eye], axis=1), d)[:, d:]
    meant = jnp.sum(eye * mean, axis=1, keepdims=True)

    gp = jnp.dot(p, cinv, preferred_element_type=jnp.float32)
    beta = jnp.sum(gp * p, axis=1, keepdims=True)
    gpaug = jnp.concatenate([-2.0 * gp, beta], axis=1)

    intra_acc = jnp.float32(0.0)
    outer_acc = jnp.float32(0.0)
    kf = jnp.float32(_K)
    for bi in range(b):
        gpb = gpaug[bi * n:(bi + 1) * n, :]

        atb = tt[bi * d:(bi + 1) * d, :] - meant
        cat = jnp.dot(cinv, atb, preferred_element_type=jnp.float32)
        alpha = jnp.sum(atb * cat, axis=0, keepdims=True)
        ataug = jnp.concatenate(
            [atb, jnp.ones((1, atb.shape[1]), jnp.float32)], axis=0)
        sit = jnp.dot(gpb, ataug, preferred_element_type=jnp.float32)
        acc_i = _bottom_k_colsum(sit, _K) + kf * alpha
        intra_acc = intra_acc + jnp.sum(acc_i * mrow[bi:bi + 1, :])

        ctb = pt[bi * d:(bi + 1) * d, :] + meant
        cct = jnp.dot(cinv, ctb, preferred_element_type=jnp.float32)
        gamma = jnp.sum(ctb * cct, axis=0, keepdims=True)
        ctaug = jnp.concatenate(
            [ctb, jnp.ones((1, ctb.shape[1]), jnp.float32)], axis=0)
        sot = jnp.dot(gpb, ctaug, preferred_element_type=jnp.float32)
        acc_o = _bottom_k_colsum(sot, _K) + kf * gamma
        outer_acc = outer_acc + jnp.sum(acc_o)

    oi_ref[0, 0] = intra_acc / count
    oo_ref[0, 0] = outer_acc / jnp.float32(bn * _K)


def kernel(outputs, targets, mask):
    bsz = targets.shape[0]
    msz = targets.shape[1]
    d = targets.shape[2]
    y_pred = outputs[:, :_MAXP]
    nsz = y_pred.shape[1]

    t_flat = targets.reshape(bsz * msz, d)
    t_t = targets.transpose(0, 2, 1).reshape(bsz * d, msz)
    mk = mask.reshape(bsz * msz, 1).astype(jnp.float32)
    mrow = mask.astype(jnp.float32)
    p_flat = y_pred.reshape(bsz * nsz, d)
    p_t = y_pred.transpose(0, 2, 1).reshape(bsz * d, nsz)

    intra, outer = pl.pallas_call(
        _loss_body,
        out_shape=[
            jax.ShapeDtypeStruct((1, 1), jnp.float32),
            jax.ShapeDtypeStruct((1, 1), jnp.float32),
        ],
        out_specs=[
            pl.BlockSpec(memory_space=pltpu.SMEM),
            pl.BlockSpec(memory_space=pltpu.SMEM),
        ],
    )(t_flat, mk, t_t, p_flat, p_t, mrow)

    intra_loss = intra[0, 0]
    outer_loss = outer[0, 0]
    return (intra_loss, intra_loss, outer_loss)


# speedup vs baseline: 3.4256x; 1.0001x over previous
import jax
import jax.numpy as jnp
from jax import lax
from jax.experimental import pallas as pl
from jax.experimental.pallas import tpu as pltpu

_K = 16
_MAXP = 512
_FMAX = 3.4e38


def _to_key(s, axis):
    b = lax.bitcast_convert_type(s, jnp.int32)
    ridx = lax.broadcasted_iota(jnp.int32, s.shape, axis)
    return lax.bitcast_convert_type((b & -512) | ridx, jnp.float32)


def _bottom_k_sum(s, k, axis):
    key = _to_key(s, axis)
    m = jnp.min(key, axis=axis, keepdims=True)
    acc = m
    for _ in range(k - 1):
        m = jnp.min(jnp.where(key > m, key, _FMAX), axis=axis, keepdims=True)
        acc = acc + m
    return acc


def _gj_inverse(a, n):
    ri = lax.broadcasted_iota(jnp.int32, (n, 1), 0)
    for kk in range(n):
        piv = a[kk:kk + 1, kk:kk + 1]
        rowk = a[kk:kk + 1, :] / piv
        colk = a[:, kk:kk + 1]
        a = jnp.where(ri == kk, rowk, a - colk * rowk)
    return a


def _loss_body(t_ref, mk_ref, tt_ref, p_ref, pt_ref, mrow_ref,
               oi_ref, oo_ref):
    t = t_ref[...]
    mk = mk_ref[...]
    tt = tt_ref[...]
    p = p_ref[...]
    pt = pt_ref[...]
    mrow = mrow_ref[...]

    bm = t.shape[0]
    bn = p.shape[0]
    d = t.shape[1]
    n = pt.shape[1]
    b = bn // n
    m = bm // b

    count = jnp.sum(mk)
    mean = jnp.sum(t * mk, axis=0, keepdims=True) / count
    am = (t - mean) * mk
    cov = lax.dot_general(
        am, am, (((0,), (0,)), ((), ())),
        preferred_element_type=jnp.float32) / (count - 1.0)
    ri = lax.broadcasted_iota(jnp.int32, (d, 1), 0)
    ci = lax.broadcasted_iota(jnp.int32, (1, d), 1)
    eye = (ci == ri).astype(jnp.float32)
    cinv = _gj_inverse(jnp.concatenate([cov, eye], axis=1), d)[:, d:]
    meant = jnp.sum(eye * mean, axis=1, keepdims=True)

    gp = jnp.dot(p, cinv, preferred_element_type=jnp.float32)
    beta = jnp.sum(gp * p, axis=1, keepdims=True)
    gpaug = jnp.concatenate([-2.0 * gp, beta], axis=1)

    a = t - mean
    ga = jnp.dot(a, cinv, preferred_element_type=jnp.float32)
    alpha = jnp.sum(ga * a, axis=1, keepdims=True)

    intra_acc = jnp.float32(0.0)
    outer_acc = jnp.float32(0.0)
    kf = jnp.float32(_K)
    for bi in range(b):
        gpb = gpaug[bi * n:(bi + 1) * n, :]

        ab = a[bi * m:(bi + 1) * m, :]
        ataug = jnp.concatenate(
            [ab, jnp.ones((m, 1), jnp.float32)], axis=1)
        gpt = jnp.dot(cinv, pt[bi * d:(bi + 1) * d, :],
                      preferred_element_type=jnp.float32)
        betat = jnp.sum(pt[bi * d:(bi + 1) * d, :] * gpt, axis=0,
                        keepdims=True)
        gpaugt = jnp.concatenate([-2.0 * gpt, betat], axis=0)
        sit = jnp.dot(ataug, gpaugt, preferred_element_type=jnp.float32)
        acc_i = _bottom_k_sum(sit, _K, 1) + kf * alpha[bi * m:(bi + 1) * m, :]
        intra_acc = intra_acc + jnp.sum(acc_i * mk[bi * m:(bi + 1) * m, :])

        ctb = pt[bi * d:(bi + 1) * d, :] + meant
        cct = jnp.dot(cinv, ctb, preferred_element_type=jnp.float32)
        gamma = jnp.sum(ctb * cct, axis=0, keepdims=True)
        ctaug = jnp.concatenate(
            [ctb, jnp.ones((1, ctb.shape[1]), jnp.float32)], axis=0)
        sot = jnp.dot(gpb, ctaug, preferred_element_type=jnp.float32)
        acc_o = _bottom_k_sum(sot, _K, 0) + kf * gamma
        outer_acc = outer_acc + jnp.sum(acc_o)

    oi_ref[0, 0] = intra_acc / count
    oo_ref[0, 0] = outer_acc / jnp.float32(bn * _K)


def kernel(outputs, targets, mask):
    bsz = targets.shape[0]
    msz = targets.shape[1]
    d = targets.shape[2]
    y_pred = outputs[:, :_MAXP]
    nsz = y_pred.shape[1]

    t_flat = targets.reshape(bsz * msz, d)
    t_t = targets.transpose(0, 2, 1).reshape(bsz * d, msz)
    mk = mask.reshape(bsz * msz, 1).astype(jnp.float32)
    mrow = mask.astype(jnp.float32)
    p_flat = y_pred.reshape(bsz * nsz, d)
    p_t = y_pred.transpose(0, 2, 1).reshape(bsz * d, nsz)

    intra, outer = pl.pallas_call(
        _loss_body,
        out_shape=[
            jax.ShapeDtypeStruct((1, 1), jnp.float32),
            jax.ShapeDtypeStruct((1, 1), jnp.float32),
        ],
        out_specs=[
            pl.BlockSpec(memory_space=pltpu.SMEM),
            pl.BlockSpec(memory_space=pltpu.SMEM),
        ],
    )(t_flat, mk, t_t, p_flat, p_t, mrow)

    intra_loss = intra[0, 0]
    outer_loss = outer[0, 0]
    return (intra_loss, intra_loss, outer_loss)
